# baseline (speedup 1.0000x reference)
import functools

import jax
import jax.numpy as jnp
from jax import lax
from jax.experimental import pallas as pl
from jax.experimental.pallas import tpu as pltpu
from jax.experimental.pallas import tpu_sc as plsc

N = 32 * 32 * 32
D = 64
H = 256
E = 8
NW = 32
CHUNK = N // NW
T = 512
P_PAD = N * 2 + E * T
G = P_PAD // T
HALF = 512
NBLK = HALF // 128


def _router_body(x_ref, wr_ref, xw_ref, i1_ref, i2_ref, g1_ref, g2_ref,
                 r1_ref, r2_ref, cnt_ref):
    x = x_ref[...]
    xw_ref[...] = jnp.concatenate(
        [x, jnp.zeros((CHUNK, 128 - D), jnp.float32)], axis=1)
    logits = lax.dot_general(x, wr_ref[...], (((1,), (0,)), ((), ())),
                             preferred_element_type=jnp.float32)
    idx = lax.broadcasted_iota(jnp.int32, logits.shape, 1)
    m1 = jnp.max(logits, axis=1, keepdims=True)
    i1 = jnp.min(jnp.where(logits == m1, idx, E), axis=1, keepdims=True)
    l2 = jnp.where(idx == i1, -jnp.inf, logits)
    m2 = jnp.max(l2, axis=1, keepdims=True)
    i2 = jnp.min(jnp.where(l2 == m2, idx, E), axis=1, keepdims=True)
    g2 = 1.0 / (1.0 + jnp.exp(m1 - m2))
    g1 = 1.0 - g2
    oh1 = (idx == i1)
    oh2 = (idx == i2)
    s = (oh1 | oh2).astype(jnp.float32)
    c_incl = s
    d = 1
    while d < CHUNK:
        z = jnp.zeros((d, E), jnp.float32)
        c_incl = c_incl + jnp.concatenate([z, c_incl[:CHUNK - d]], axis=0)
        d *= 2
    c_excl = c_incl - s
    r1 = jnp.sum(jnp.where(oh1, c_excl, 0.0), axis=1, keepdims=True)
    r2 = jnp.sum(jnp.where(oh2, c_excl, 0.0), axis=1, keepdims=True)
    i1_ref[...] = i1
    i2_ref[...] = i2
    g1_ref[...] = g1
    g2_ref[...] = g2
    r1_ref[...] = r1.astype(jnp.int32)
    r2_ref[...] = r2.astype(jnp.int32)
    cnt_ref[...] = jnp.sum(s, axis=0).astype(jnp.int32)[None, None, :]


def _router(states, W_router):
    col_i = pl.BlockSpec((CHUNK, 1), lambda i: (i, 0))
    return pl.pallas_call(
        _router_body,
        grid=(NW,),
        in_specs=[
            pl.BlockSpec((CHUNK, D), lambda i: (i, 0)),
            pl.BlockSpec((D, E), lambda i: (0, 0)),
        ],
        out_specs=[pl.BlockSpec((CHUNK, 128), lambda i: (i, 0)),
                   col_i, col_i, col_i, col_i, col_i, col_i,
                   pl.BlockSpec((1, 1, E), lambda i: (i, 0, 0))],
        out_shape=[
            jax.ShapeDtypeStruct((N, 128), jnp.float32),
            jax.ShapeDtypeStruct((N, 1), jnp.int32),
            jax.ShapeDtypeStruct((N, 1), jnp.int32),
            jax.ShapeDtypeStruct((N, 1), jnp.float32),
            jax.ShapeDtypeStruct((N, 1), jnp.float32),
            jax.ShapeDtypeStruct((N, 1), jnp.int32),
            jax.ShapeDtypeStruct((N, 1), jnp.int32),
            jax.ShapeDtypeStruct((NW, 1, E), jnp.int32),
        ],
        compiler_params=pltpu.CompilerParams(
            dimension_semantics=("arbitrary",)),
    )(states, W_router)


def _slot_body(i1_ref, i2_ref, r1_ref, r2_ref, cs_ref, s1_ref, s2_ref):
    i1 = i1_ref[...]
    i2 = i2_ref[...]
    cs = cs_ref[...].reshape(1, E)
    b1 = jnp.zeros_like(i1)
    b2 = jnp.zeros_like(i2)
    for e in range(E):
        ce = cs[:, e:e + 1]
        b1 = jnp.where(i1 == e, ce, b1)
        b2 = jnp.where(i2 == e, ce, b2)
    s1_ref[...] = b1 + r1_ref[...]
    s2_ref[...] = b2 + r2_ref[...]


def _slots(i1, i2, r1, r2, chunk_start):
    col_i = pl.BlockSpec((CHUNK, 1), lambda i: (i, 0))
    return pl.pallas_call(
        _slot_body,
        grid=(NW,),
        in_specs=[col_i, col_i, col_i, col_i,
                  pl.BlockSpec((1, 1, E), lambda i: (i, 0, 0))],
        out_specs=[col_i, col_i],
        out_shape=[
            jax.ShapeDtypeStruct((N, 1), jnp.int32),
            jax.ShapeDtypeStruct((N, 1), jnp.int32),
        ],
        compiler_params=pltpu.CompilerParams(
            dimension_semantics=("arbitrary",)),
    )(i1, i2, r1, r2, chunk_start)


def _dispatch_body(x_hbm, s1_hbm, s2_hbm, xs_hbm, x_v, i1_v, i2_v, sem):
    wid = lax.axis_index("s") * 2 + lax.axis_index("c")
    pltpu.sync_copy(s1_hbm.at[pl.ds(wid * (CHUNK // 128), CHUNK // 128)],
                    i1_v)
    pltpu.sync_copy(s2_hbm.at[pl.ds(wid * (CHUNK // 128), CHUNK // 128)],
                    i2_v)
    for h in range(CHUNK // HALF):
        pltpu.sync_copy(x_hbm.at[pl.ds(wid * CHUNK + h * HALF, HALF)], x_v)
        cps = []
        for b in range(NBLK):
            src = x_v.at[pl.ds(b * 128, 128)]
            cps.append(pltpu.make_async_copy(
                src, xs_hbm.at[i1_v.at[h * NBLK + b]], sem))
            cps.append(pltpu.make_async_copy(
                src, xs_hbm.at[i2_v.at[h * NBLK + b]], sem))
        for cp in cps:
            cp.start()
        for cp in cps:
            cp.wait()


def _ffn_body(g_ref, x_ref, w1_ref, b1_ref, w2_ref, b2_ref, y_ref):
    x = x_ref[:, :D].astype(jnp.bfloat16)
    h = lax.dot_general(x, w1_ref[0], (((1,), (0,)), ((), ())),
                        preferred_element_type=jnp.float32)
    h = jax.nn.gelu(h + b1_ref[0]).astype(jnp.bfloat16)
    y = lax.dot_general(h, w2_ref[0], (((1,), (0,)), ((), ())),
                        preferred_element_type=jnp.float32)
    y_ref[...] = jnp.concatenate(
        [y + b2_ref[0], jnp.zeros((T, 128 - D), jnp.float32)], axis=1)


def _ffn(gidx, xs, W1, b1, W2, b2):
    spec = pltpu.PrefetchScalarGridSpec(
        num_scalar_prefetch=1,
        grid=(G,),
        in_specs=[
            pl.BlockSpec((T, 128), lambda i, g: (i, 0)),
            pl.BlockSpec((1, D, H), lambda i, g: (g[i], 0, 0)),
            pl.BlockSpec((1, 1, H), lambda i, g: (g[i], 0, 0)),
            pl.BlockSpec((1, H, D), lambda i, g: (g[i], 0, 0)),
            pl.BlockSpec((1, 1, D), lambda i, g: (g[i], 0, 0)),
        ],
        out_specs=pl.BlockSpec((T, 128), lambda i, g: (i, 0)),
    )
    return pl.pallas_call(
        _ffn_body,
        grid_spec=spec,
        out_shape=jax.ShapeDtypeStruct((P_PAD, 128), jnp.float32),
        compiler_params=pltpu.CompilerParams(
            dimension_semantics=("arbitrary",)),
    )(gidx, xs, W1, b1, W2, b2)


def _gather_body(ys_hbm, s1_hbm, s2_hbm, y1_hbm, y2_hbm, y_v, i1_v, i2_v,
                 sem):
    wid = lax.axis_index("s") * 2 + lax.axis_index("c")
    pltpu.sync_copy(s1_hbm.at[pl.ds(wid * (CHUNK // 128), CHUNK // 128)],
                    i1_v)
    pltpu.sync_copy(s2_hbm.at[pl.ds(wid * (CHUNK // 128), CHUNK // 128)],
                    i2_v)
    for iv, out_hbm in ((i1_v, y1_hbm), (i2_v, y2_hbm)):
        for h in range(CHUNK // HALF):
            cps = []
            for b in range(NBLK):
                cps.append(pltpu.make_async_copy(
                    ys_hbm.at[iv.at[h * NBLK + b]],
                    y_v.at[pl.ds(b * 128, 128)], sem))
            for cp in cps:
                cp.start()
            for cp in cps:
                cp.wait()
            pltpu.sync_copy(y_v,
                            out_hbm.at[pl.ds(wid * CHUNK + h * HALF, HALF)])


@functools.cache
def _sc_kernels():
    mesh = plsc.VectorSubcoreMesh(core_axis_name="c", subcore_axis_name="s")
    scratch = (
        pltpu.VMEM((HALF, 128), jnp.float32),
        pltpu.VMEM((CHUNK // 128, 128), jnp.int32),
        pltpu.VMEM((CHUNK // 128, 128), jnp.int32),
        pltpu.SemaphoreType.DMA,
    )
    dispatch = pl.kernel(
        _dispatch_body,
        out_type=jax.ShapeDtypeStruct((P_PAD, 128), jnp.float32),
        mesh=mesh, scratch_types=scratch)
    gather = pl.kernel(
        _gather_body,
        out_type=(jax.ShapeDtypeStruct((N, 128), jnp.float32),
                  jax.ShapeDtypeStruct((N, 128), jnp.float32)),
        mesh=mesh, scratch_types=scratch)
    return dispatch, gather


CBLK = 2048


def _combine_body(x_ref, y1_ref, y2_ref, g1_ref, g2_ref, o_ref):
    o_ref[...] = (x_ref[...] + g1_ref[...] * y1_ref[:, :D]
                  + g2_ref[...] * y2_ref[:, :D])


def _combine(states, y1, y2, g1, g2):
    row = pl.BlockSpec((CBLK, D), lambda i: (i, 0))
    wide = pl.BlockSpec((CBLK, 128), lambda i: (i, 0))
    col = pl.BlockSpec((CBLK, 1), lambda i: (i, 0))
    return pl.pallas_call(
        _combine_body,
        grid=(N // CBLK,),
        in_specs=[row, wide, wide, col, col],
        out_specs=row,
        out_shape=jax.ShapeDtypeStruct((N, D), jnp.float32),
        compiler_params=pltpu.CompilerParams(
            dimension_semantics=("arbitrary",)),
    )(states, y1, y2, g1, g2)


@jax.jit
def kernel(states, W_router, W1, b1, W2, b2):
    xw, i1, i2, g1, g2, r1, r2, counts3 = _router(states, W_router)
    counts = counts3.reshape(NW, E)
    totals = jnp.sum(counts, axis=0)
    pad_tot = ((totals + T - 1) // T) * T
    ends = jnp.cumsum(pad_tot)
    base = ends - pad_tot
    cum_chunks = jnp.cumsum(counts, axis=0) - counts
    chunk_start = (base[None, :] + cum_chunks).astype(jnp.int32)
    tile_starts = jnp.arange(G, dtype=jnp.int32) * T
    gidx = jnp.sum((tile_starts[:, None] >= ends[None, :]).astype(jnp.int32),
                   axis=1)
    gidx = jnp.minimum(gidx, E - 1).astype(jnp.int32)

    s1, s2 = _slots(i1, i2, r1, r2, chunk_start.reshape(NW, 1, E))
    s1_2d = s1.reshape(N // 128, 128)
    s2_2d = s2.reshape(N // 128, 128)
    dispatch, gather = _sc_kernels()
    xs = dispatch(xw, s1_2d, s2_2d)
    ys = _ffn(gidx, xs, W1.astype(jnp.bfloat16), b1.reshape(E, 1, H),
              W2.astype(jnp.bfloat16), b2.reshape(E, 1, D))
    y1, y2 = gather(ys, s1_2d, s2_2d)
    return _combine(states, y1, y2, g1, g2)


# speedup vs baseline: 1.2269x; 1.2269x over previous
import functools

import jax
import jax.numpy as jnp
from jax import lax
from jax.experimental import pallas as pl
from jax.experimental.pallas import tpu as pltpu
from jax.experimental.pallas import tpu_sc as plsc

N = 32 * 32 * 32
D = 64
H = 256
E = 8
NW = 32
CHUNK = N // NW
T = 2048
P_PAD = N * 2 + E * T
G = P_PAD // T
HALF = 512
NBLK = HALF // 128


def _router_body(x_ref, wr_ref, xw_ref, i1_ref, i2_ref, g1_ref, g2_ref,
                 r1_ref, r2_ref, cnt_ref):
    x = x_ref[...]
    xw_ref[...] = jnp.concatenate(
        [x, jnp.zeros((CHUNK, 128 - D), jnp.float32)], axis=1)
    logits = lax.dot_general(x, wr_ref[...], (((1,), (0,)), ((), ())),
                             preferred_element_type=jnp.float32)
    idx = lax.broadcasted_iota(jnp.int32, logits.shape, 1)
    m1 = jnp.max(logits, axis=1, keepdims=True)
    i1 = jnp.min(jnp.where(logits == m1, idx, E), axis=1, keepdims=True)
    l2 = jnp.where(idx == i1, -jnp.inf, logits)
    m2 = jnp.max(l2, axis=1, keepdims=True)
    i2 = jnp.min(jnp.where(l2 == m2, idx, E), axis=1, keepdims=True)
    g2 = 1.0 / (1.0 + jnp.exp(m1 - m2))
    g1 = 1.0 - g2
    oh1 = (idx == i1)
    oh2 = (idx == i2)
    s = (oh1 | oh2).astype(jnp.float32)
    c_incl = s
    d = 1
    while d < CHUNK:
        z = jnp.zeros((d, E), jnp.float32)
        c_incl = c_incl + jnp.concatenate([z, c_incl[:CHUNK - d]], axis=0)
        d *= 2
    c_excl = c_incl - s
    r1 = jnp.sum(jnp.where(oh1, c_excl, 0.0), axis=1, keepdims=True)
    r2 = jnp.sum(jnp.where(oh2, c_excl, 0.0), axis=1, keepdims=True)
    i1_ref[...] = i1
    i2_ref[...] = i2
    g1_ref[...] = g1
    g2_ref[...] = g2
    r1_ref[...] = r1.astype(jnp.int32)
    r2_ref[...] = r2.astype(jnp.int32)
    cnt_ref[...] = jnp.sum(s, axis=0).astype(jnp.int32)[None, None, :]


def _router(states, W_router):
    col_i = pl.BlockSpec((CHUNK, 1), lambda i: (i, 0))
    return pl.pallas_call(
        _router_body,
        grid=(NW,),
        in_specs=[
            pl.BlockSpec((CHUNK, D), lambda i: (i, 0)),
            pl.BlockSpec((D, E), lambda i: (0, 0)),
        ],
        out_specs=[pl.BlockSpec((CHUNK, 128), lambda i: (i, 0)),
                   col_i, col_i, col_i, col_i, col_i, col_i,
                   pl.BlockSpec((1, 1, E), lambda i: (i, 0, 0))],
        out_shape=[
            jax.ShapeDtypeStruct((N, 128), jnp.float32),
            jax.ShapeDtypeStruct((N, 1), jnp.int32),
            jax.ShapeDtypeStruct((N, 1), jnp.int32),
            jax.ShapeDtypeStruct((N, 1), jnp.float32),
            jax.ShapeDtypeStruct((N, 1), jnp.float32),
            jax.ShapeDtypeStruct((N, 1), jnp.int32),
            jax.ShapeDtypeStruct((N, 1), jnp.int32),
            jax.ShapeDtypeStruct((NW, 1, E), jnp.int32),
        ],
        compiler_params=pltpu.CompilerParams(
            dimension_semantics=("arbitrary",)),
    )(states, W_router)


def _slot_body(i1_ref, i2_ref, r1_ref, r2_ref, cs_ref, s1_ref, s2_ref):
    i1 = i1_ref[...]
    i2 = i2_ref[...]
    cs = cs_ref[...].reshape(1, E)
    b1 = jnp.zeros_like(i1)
    b2 = jnp.zeros_like(i2)
    for e in range(E):
        ce = cs[0:1, e:e + 1]
        b1 = jnp.where(i1 == e, ce, b1)
        b2 = jnp.where(i2 == e, ce, b2)
    s1_ref[...] = b1 + r1_ref[...]
    s2_ref[...] = b2 + r2_ref[...]


def _slots(i1, i2, r1, r2, chunk_start):
    blk = pl.BlockSpec((CHUNK // 128, 128), lambda i: (i, 0))
    return pl.pallas_call(
        _slot_body,
        grid=(NW,),
        in_specs=[blk, blk, blk, blk,
                  pl.BlockSpec((1, 1, E), lambda i: (i, 0, 0))],
        out_specs=[blk, blk],
        out_shape=[
            jax.ShapeDtypeStruct((N // 128, 128), jnp.int32),
            jax.ShapeDtypeStruct((N // 128, 128), jnp.int32),
        ],
        compiler_params=pltpu.CompilerParams(
            dimension_semantics=("arbitrary",)),
    )(i1, i2, r1, r2, chunk_start)


def _dispatch_body(x_hbm, s1_hbm, s2_hbm, xs_hbm, x_v, i1_v, i2_v, sem):
    wid = lax.axis_index("s") * 2 + lax.axis_index("c")
    pltpu.sync_copy(s1_hbm.at[pl.ds(wid * (CHUNK // 128), CHUNK // 128)],
                    i1_v)
    pltpu.sync_copy(s2_hbm.at[pl.ds(wid * (CHUNK // 128), CHUNK // 128)],
                    i2_v)
    for h in range(CHUNK // HALF):
        pltpu.sync_copy(x_hbm.at[pl.ds(wid * CHUNK + h * HALF, HALF)], x_v)
        cps = []
        for b in range(NBLK):
            src = x_v.at[pl.ds(b * 128, 128)]
            cps.append(pltpu.make_async_copy(
                src, xs_hbm.at[i1_v.at[h * NBLK + b]], sem))
            cps.append(pltpu.make_async_copy(
                src, xs_hbm.at[i2_v.at[h * NBLK + b]], sem))
        for cp in cps:
            cp.start()
        for cp in cps:
            cp.wait()


def _ffn_body(g_ref, x_ref, w1_ref, b1_ref, w2_ref, b2_ref, y_ref):
    x = x_ref[:, :D].astype(jnp.bfloat16)
    h = lax.dot_general(x, w1_ref[0], (((1,), (0,)), ((), ())),
                        preferred_element_type=jnp.float32)
    h = jax.nn.gelu(h + b1_ref[0]).astype(jnp.bfloat16)
    y = lax.dot_general(h, w2_ref[0], (((1,), (0,)), ((), ())),
                        preferred_element_type=jnp.float32)
    y_ref[...] = jnp.concatenate(
        [y + b2_ref[0], jnp.zeros((T, 128 - D), jnp.float32)], axis=1)


def _ffn(gidx, xs, W1, b1, W2, b2):
    spec = pltpu.PrefetchScalarGridSpec(
        num_scalar_prefetch=1,
        grid=(G,),
        in_specs=[
            pl.BlockSpec((T, 128), lambda i, g: (i, 0)),
            pl.BlockSpec((1, D, H), lambda i, g: (g[i], 0, 0)),
            pl.BlockSpec((1, 1, H), lambda i, g: (g[i], 0, 0)),
            pl.BlockSpec((1, H, D), lambda i, g: (g[i], 0, 0)),
            pl.BlockSpec((1, 1, D), lambda i, g: (g[i], 0, 0)),
        ],
        out_specs=pl.BlockSpec((T, 128), lambda i, g: (i, 0)),
    )
    return pl.pallas_call(
        _ffn_body,
        grid_spec=spec,
        out_shape=jax.ShapeDtypeStruct((P_PAD, 128), jnp.float32),
        compiler_params=pltpu.CompilerParams(
            dimension_semantics=("arbitrary",)),
    )(gidx, xs, W1, b1, W2, b2)


def _gather_body(ys_hbm, s1_hbm, s2_hbm, y1_hbm, y2_hbm, y_v, i1_v, i2_v,
                 sem):
    wid = lax.axis_index("s") * 2 + lax.axis_index("c")
    pltpu.sync_copy(s1_hbm.at[pl.ds(wid * (CHUNK // 128), CHUNK // 128)],
                    i1_v)
    pltpu.sync_copy(s2_hbm.at[pl.ds(wid * (CHUNK // 128), CHUNK // 128)],
                    i2_v)
    for iv, out_hbm in ((i1_v, y1_hbm), (i2_v, y2_hbm)):
        for h in range(CHUNK // HALF):
            cps = []
            for b in range(NBLK):
                cps.append(pltpu.make_async_copy(
                    ys_hbm.at[iv.at[h * NBLK + b]],
                    y_v.at[pl.ds(b * 128, 128)], sem))
            for cp in cps:
                cp.start()
            for cp in cps:
                cp.wait()
            pltpu.sync_copy(y_v,
                            out_hbm.at[pl.ds(wid * CHUNK + h * HALF, HALF)])


@functools.cache
def _sc_kernels():
    mesh = plsc.VectorSubcoreMesh(core_axis_name="c", subcore_axis_name="s")
    scratch = (
        pltpu.VMEM((HALF, 128), jnp.float32),
        pltpu.VMEM((CHUNK // 128, 128), jnp.int32),
        pltpu.VMEM((CHUNK // 128, 128), jnp.int32),
        pltpu.SemaphoreType.DMA,
    )
    dispatch = pl.kernel(
        _dispatch_body,
        out_type=jax.ShapeDtypeStruct((P_PAD, 128), jnp.float32),
        mesh=mesh, scratch_types=scratch)
    gather = pl.kernel(
        _gather_body,
        out_type=(jax.ShapeDtypeStruct((N, 128), jnp.float32),
                  jax.ShapeDtypeStruct((N, 128), jnp.float32)),
        mesh=mesh, scratch_types=scratch)
    return dispatch, gather


CBLK = 2048


def _combine_body(x_ref, y1_ref, y2_ref, g1_ref, g2_ref, o_ref):
    o_ref[...] = (x_ref[...] + g1_ref[...] * y1_ref[:, :D]
                  + g2_ref[...] * y2_ref[:, :D])


def _combine(states, y1, y2, g1, g2):
    row = pl.BlockSpec((CBLK, D), lambda i: (i, 0))
    wide = pl.BlockSpec((CBLK, 128), lambda i: (i, 0))
    col = pl.BlockSpec((CBLK, 1), lambda i: (i, 0))
    return pl.pallas_call(
        _combine_body,
        grid=(N // CBLK,),
        in_specs=[row, wide, wide, col, col],
        out_specs=row,
        out_shape=jax.ShapeDtypeStruct((N, D), jnp.float32),
        compiler_params=pltpu.CompilerParams(
            dimension_semantics=("arbitrary",)),
    )(states, y1, y2, g1, g2)


@jax.jit
def kernel(states, W_router, W1, b1, W2, b2):
    xw, i1, i2, g1, g2, r1, r2, counts3 = _router(states, W_router)
    counts = counts3.reshape(NW, E)
    totals = jnp.sum(counts, axis=0)
    pad_tot = ((totals + T - 1) // T) * T
    ends = jnp.cumsum(pad_tot)
    base = ends - pad_tot
    cum_chunks = jnp.cumsum(counts, axis=0) - counts
    chunk_start = (base[None, :] + cum_chunks).astype(jnp.int32)
    tile_starts = jnp.arange(G, dtype=jnp.int32) * T
    gidx = jnp.sum((tile_starts[:, None] >= ends[None, :]).astype(jnp.int32),
                   axis=1)
    gidx = jnp.minimum(gidx, E - 1).astype(jnp.int32)

    q = N // 128
    s1_2d, s2_2d = _slots(i1.reshape(q, 128), i2.reshape(q, 128),
                          r1.reshape(q, 128), r2.reshape(q, 128),
                          chunk_start.reshape(NW, 1, E))
    dispatch, gather = _sc_kernels()
    xs = dispatch(xw, s1_2d, s2_2d)
    ys = _ffn(gidx, xs, W1.astype(jnp.bfloat16), b1.reshape(E, 1, H),
              W2.astype(jnp.bfloat16), b2.reshape(E, 1, D))
    y1, y2 = gather(ys, s1_2d, s2_2d)
    return _combine(states, y1, y2, g1, g2)
